# Initial kernel scaffold; baseline (speedup 1.0000x reference)
import jax
import jax.numpy as jnp
from jax.experimental import pallas as pl


def kernel(txt_input, table):
    raise NotImplementedError("write your pallas kernel here")


# speedup vs baseline: 1.6955x; 1.6955x over previous
import jax
import jax.numpy as jnp
from jax import lax
from jax.experimental import pallas as pl
from jax.experimental.pallas import tpu as pltpu
from jax.experimental.pallas import tpu_sc as plsc

_VOCAB = 1000000
_D = 16
_B = 16384
_SEQ = 50

_NC = 2
_NS = 16
_NW = _NC * _NS
_B_PER_W = _B // _NW
_CHUNK_C = 64
_N_CHUNK = _B_PER_W // _CHUNK_C
_IDX_PER_CHUNK = _CHUNK_C * _SEQ
_IDX_W = 128
_IDX_ROWS = _IDX_PER_CHUNK // _IDX_W
_IDX_ROWS_W = _B_PER_W * _SEQ // _IDX_W


def _encoder_body(idx_hbm, table_hbm, out_hbm, idx_v, rows_v, out_v, sems):
    wid = lax.axis_index("s") * _NC + lax.axis_index("c")

    pltpu.sync_copy(idx_hbm.at[pl.ds(wid * _IDX_ROWS_W, _IDX_ROWS_W)], idx_v)

    def fire(g, p):
        def body(j, carry):
            pltpu.make_async_copy(
                table_hbm.at[idx_v.at[g * _IDX_ROWS + j]],
                rows_v.at[p].at[pl.ds(j * _IDX_W, _IDX_W)],
                sems.at[p],
            ).start()
            return carry

        lax.fori_loop(0, _IDX_ROWS, body, 0, unroll=False)

    def drain(p):
        pltpu.make_async_copy(
            table_hbm.at[pl.ds(0, _IDX_PER_CHUNK)], rows_v.at[p], sems.at[p]
        ).wait()

    def compute_store(g, p):
        r = rows_v

        def cbody(c, carry):
            base = c * _SEQ
            a0 = r[p, base, :]
            a1 = r[p, base + 1, :]
            a2 = r[p, base + 2, :]
            a3 = r[p, base + 3, :]
            for l in range(4, _SEQ - 2, 4):
                a0 = a0 + r[p, base + l, :]
                a1 = a1 + r[p, base + l + 1, :]
                a2 = a2 + r[p, base + l + 2, :]
                a3 = a3 + r[p, base + l + 3, :]
            a0 = a0 + r[p, base + _SEQ - 2, :]
            a1 = a1 + r[p, base + _SEQ - 1, :]
            out_v[c, :] = ((a0 + a1) + (a2 + a3)) * jnp.float32(1.0 / _SEQ)
            return carry

        lax.fori_loop(0, _CHUNK_C, cbody, 0, unroll=False)
        out_base = wid * _B_PER_W + g * _CHUNK_C
        pltpu.sync_copy(out_v, out_hbm.at[pl.ds(out_base, _CHUNK_C)])

    fire(0, 0)
    for g in range(_N_CHUNK):
        p = g % 2
        if g + 1 < _N_CHUNK:
            fire(g + 1, 1 - p)
        drain(p)
        compute_store(g, p)


def kernel(txt_input, table):
    idx2d = txt_input.reshape(_B * _SEQ // _IDX_W, _IDX_W)
    mesh = plsc.VectorSubcoreMesh(core_axis_name="c", subcore_axis_name="s")
    run = pl.kernel(
        _encoder_body,
        out_type=jax.ShapeDtypeStruct((_B, _D), jnp.float32),
        mesh=mesh,
        scratch_types=[
            pltpu.VMEM((_IDX_ROWS_W, _IDX_W), jnp.int32),
            pltpu.VMEM((2, _IDX_PER_CHUNK, _D), jnp.float32),
            pltpu.VMEM((_CHUNK_C, _D), jnp.float32),
            pltpu.SemaphoreType.DMA((2,)),
        ],
        compiler_params=pltpu.CompilerParams(use_tc_tiling_on_sc=False),
    )
    return run(idx2d, table)
